# Initial kernel scaffold; baseline (speedup 1.0000x reference)
import jax
import jax.numpy as jnp
from jax.experimental import pallas as pl


def kernel(z_e, embedding):
    raise NotImplementedError("write your pallas kernel here")


# speedup vs baseline: 1.2071x; 1.2071x over previous
import jax
import jax.numpy as jnp
from jax.experimental import pallas as pl
from jax.experimental.pallas import tpu as pltpu

_NUM_EMB = 1024
_EMB_DIM = 64
_BETA = 0.25
_EPS = 1e-10


def _vq_kernel(z_ref, emb_ref, enc_ref, zq_ref, loss_ref, perp_ref,
               counts_ref, ssq_ref):
    i = pl.program_id(0)
    nsteps = pl.num_programs(0)

    @pl.when(i == 0)
    def _init():
        counts_ref[...] = jnp.zeros_like(counts_ref)
        ssq_ref[0, 0] = 0.0

    x = z_ref[0]
    xt = x.T
    emb = emb_ref[...]

    inner = jax.lax.dot_general(
        xt, emb, (((1,), (1,)), ((), ())),
        preferred_element_type=jnp.float32)
    flat_l2 = jnp.sum(xt * xt, axis=1, keepdims=True)
    emb_l2 = jnp.sum(emb * emb, axis=1)[None, :]
    dist = (flat_l2 + emb_l2) - 2.0 * inner
    iota = jax.lax.broadcasted_iota(jnp.int32, (xt.shape[0], _NUM_EMB), 1)
    dmin = jnp.min(dist, axis=1, keepdims=True)
    idx = jnp.min(jnp.where(dist == dmin, iota, jnp.int32(2 ** 30)),
                  axis=1)
    onehot = (iota == idx[:, None]).astype(jnp.float32)
    enc_ref[...] = onehot

    zq_rows = jax.lax.dot_general(
        onehot, emb, (((1,), (0,)), ((), ())),
        preferred_element_type=jnp.float32)
    zq_ref[0] = zq_rows.T

    diff = zq_rows - xt
    ssq_ref[0, 0] += jnp.sum(diff * diff)
    counts_ref[...] += jnp.sum(onehot, axis=0, keepdims=True)

    @pl.when(i == nsteps - 1)
    def _fin():
        n_vec = nsteps * 1024
        mean_sq = ssq_ref[0, 0] / (n_vec * _EMB_DIM)
        loss_ref[...] = jnp.full((1, 1), _BETA * mean_sq + mean_sq, jnp.float32)
        p = counts_ref[...] * (1.0 / n_vec)
        plogp = p * jnp.log(p + _EPS)
        perp_ref[...] = jnp.exp(-jnp.sum(plogp, axis=1, keepdims=True))


def kernel(z_e, embedding):
    B, C, H, W = z_e.shape
    HW = H * W
    z3 = z_e.reshape(B, C, HW)
    enc, zq3, loss, perp = pl.pallas_call(
        _vq_kernel,
        grid=(B,),
        in_specs=[
            pl.BlockSpec((1, C, HW), lambda i: (i, 0, 0)),
            pl.BlockSpec((_NUM_EMB, _EMB_DIM), lambda i: (0, 0)),
        ],
        out_specs=[
            pl.BlockSpec((HW, _NUM_EMB), lambda i: (i, 0)),
            pl.BlockSpec((1, C, HW), lambda i: (i, 0, 0)),
            pl.BlockSpec((1, 1), lambda i: (0, 0)),
            pl.BlockSpec((1, 1), lambda i: (0, 0)),
        ],
        out_shape=[
            jax.ShapeDtypeStruct((B * HW, _NUM_EMB), jnp.float32),
            jax.ShapeDtypeStruct((B, C, HW), jnp.float32),
            jax.ShapeDtypeStruct((1, 1), jnp.float32),
            jax.ShapeDtypeStruct((1, 1), jnp.float32),
        ],
        scratch_shapes=[
            pltpu.VMEM((1, _NUM_EMB), jnp.float32),
            pltpu.SMEM((1, 1), jnp.float32),
        ],
    )(z3, embedding)
    zq = zq3.reshape(B, C, H, W)
    return (loss[0, 0], zq, perp[0, 0], enc)
